# Initial kernel scaffold; baseline (speedup 1.0000x reference)
import jax
import jax.numpy as jnp
from jax.experimental import pallas as pl


def kernel(h, edge_index):
    raise NotImplementedError("write your pallas kernel here")


# speedup vs baseline: 2.0814x; 2.0814x over previous
import dataclasses
import functools

import jax
import jax.numpy as jnp
from jax import lax
from jax.experimental import pallas as pl
from jax.experimental.pallas import tpu as pltpu
from jax.experimental.pallas import tpu_sc as plsc

D = 128
L = 16
NC, NS = 2, 16
NW = NC * NS
C = 128


@functools.cache
def _dot_kernel(E_pad):
    per_w = E_pad // NW
    n_chunks = per_w // C

    mesh = plsc.VectorSubcoreMesh(core_axis_name="c", subcore_axis_name="s")

    cp = pltpu.CompilerParams()
    if "needs_layout_passes" in pltpu.CompilerParams.__dataclass_fields__:
        cp = dataclasses.replace(cp, needs_layout_passes=False)

    @functools.partial(
        pl.kernel,
        mesh=mesh,
        compiler_params=cp,
        out_type=jax.ShapeDtypeStruct((E_pad,), jnp.float32),
        scratch_types=[
            pltpu.VMEM((C,), jnp.int32),
            pltpu.VMEM((C,), jnp.int32),
            pltpu.VMEM((C, D), jnp.float32),
            pltpu.VMEM((C, D), jnp.float32),
            pltpu.VMEM((C,), jnp.float32),
            pltpu.SemaphoreType.DMA,
            pltpu.SemaphoreType.DMA,
        ],
    )
    def k(h_hbm, src_hbm, dst_hbm, out_hbm,
          sidx, didx, srows, drows, ovec, sem_s, sem_d):
        wid = lax.axis_index("s") * NC + lax.axis_index("c")
        base = wid * per_w

        @pl.loop(0, n_chunks)
        def _chunk(t):
            b = base + t * C
            pltpu.sync_copy(src_hbm.at[pl.ds(b, C)], sidx)
            pltpu.sync_copy(dst_hbm.at[pl.ds(b, C)], didx)
            cps = pltpu.async_copy(h_hbm.at[sidx], srows, sem_s)
            cpd = pltpu.async_copy(h_hbm.at[didx], drows, sem_d)
            cps.wait()
            cpd.wait()

            lane = lax.iota(jnp.int32, L)
            last = lane == (L - 1)

            @pl.loop(0, C // L)
            def _grp(g):
                e0 = g * L
                e0v = jnp.full((L,), e0, jnp.int32)
                for j in range(L):
                    e = e0 + j
                    p = srows[e, pl.ds(0, L)] * drows[e, pl.ds(0, L)]
                    for kk in range(1, D // L):
                        p = p + (srows[e, pl.ds(kk * L, L)]
                                 * drows[e, pl.ds(kk * L, L)])
                    ps = lax.cumsum(p, axis=0)
                    plsc.store_scatter(ovec, [e0v + j], ps, mask=last)

            pltpu.sync_copy(ovec, out_hbm.at[pl.ds(b, C)])

    return k


def kernel(h, edge_index):
    E = edge_index.shape[1]
    src = edge_index[0].astype(jnp.int32)
    dst = edge_index[1].astype(jnp.int32)

    step = NW * C
    E_pad = ((E + step - 1) // step) * step
    if E_pad != E:
        pad = E_pad - E
        zeros = jnp.zeros((pad,), jnp.int32)
        src = jnp.concatenate([src, zeros])
        dst = jnp.concatenate([dst, zeros])

    out = _dot_kernel(E_pad)(h, src, dst)
    return out[:E].reshape(E, 1)
